# baseline (speedup 1.0000x reference)
import functools

import jax
import jax.numpy as jnp
from jax import lax
from jax.experimental import pallas as pl
from jax.experimental.pallas import tpu as pltpu
from jax.experimental.pallas import tpu_sc as plsc

N = 10000
E = 320000
IN = 128
H = 8
OUT = 16

NC = 2
NS = 16
NW = NC * NS
B = 128
NB = E // B
CB = 10
NCHUNKS = NB // CB
CE = CB * B
BASE_CHUNKS = NCHUNKS // NW
EXTRA = NCHUNKS - BASE_CHUNKS * NW


def _tc_tables(x_ref, w_ref, sa_ref, sb_ref, a_ref, b_ref):
    wa = jnp.dot(w_ref[...], sa_ref[...], preferred_element_type=jnp.float32)
    wb = jnp.dot(w_ref[...], sb_ref[...], preferred_element_type=jnp.float32)
    x = x_ref[...]
    a_ref[...] = jnp.dot(x, wa, preferred_element_type=jnp.float32)
    b_ref[...] = jnp.dot(x, wb, preferred_element_type=jnp.float32)


def _make_tables(x, W, S_a, S_b):
    return pl.pallas_call(
        _tc_tables,
        out_shape=(
            jax.ShapeDtypeStruct((N, 16), jnp.float32),
            jax.ShapeDtypeStruct((N, 16), jnp.float32),
        ),
    )(x, W, S_a, S_b)


_sc_mesh = plsc.VectorSubcoreMesh(
    core_axis_name="c", subcore_axis_name="s", num_cores=NC, num_subcores=NS
)


@functools.partial(
    pl.kernel,
    out_type=jax.ShapeDtypeStruct((E * H,), jnp.float32),
    mesh=_sc_mesh,
    scratch_types=[
        pltpu.VMEM((2, CB, B), jnp.int32),
        pltpu.VMEM((2, CB, B), jnp.int32),
        pltpu.VMEM((2, CE, 16), jnp.float32),
        pltpu.VMEM((2, CE, 16), jnp.float32),
        pltpu.VMEM((CE * H,), jnp.float32),
        pltpu.SemaphoreType.DMA,
        pltpu.SemaphoreType.DMA,
        pltpu.SemaphoreType.DMA,
        pltpu.SemaphoreType.DMA,
        pltpu.SemaphoreType.DMA,
    ],
    compiler_params=pltpu.CompilerParams(
        use_tc_tiling_on_sc=False, needs_layout_passes=False),
)
def _sc_edges(ta_hbm, tb_hbm, head_hbm, tail_hbm, out_hbm,
              hidx, tidx, ra, rb, ob,
              sem_a0, sem_a1, sem_b0, sem_b1, sem_out):
    wid = lax.axis_index("s") * NC + lax.axis_index("c")
    n = jnp.where(wid < EXTRA, BASE_CHUNKS + 1, BASE_CHUNKS)
    lane = lax.iota(jnp.int32, 16)
    lane_lo = lane < 8
    svec = (lane & 7) * B + lax.shift_right_logical(lane, 3)
    sem_a = (sem_a0, sem_a1)
    sem_b = (sem_b0, sem_b1)

    def fire(c, p):
        @pl.when(c < n)
        def _():
            rbase = (wid + c * NW) * CB
            pltpu.sync_copy(head_hbm.at[pl.ds(rbase, CB)], hidx.at[p])
            pltpu.sync_copy(tail_hbm.at[pl.ds(rbase, CB)], tidx.at[p])
            for kk in range(CB):
                pltpu.async_copy(ta_hbm.at[hidx.at[p].at[kk]],
                                 ra.at[p].at[pl.ds(kk * B, B)], sem_a[p])
                pltpu.async_copy(tb_hbm.at[tidx.at[p].at[kk]],
                                 rb.at[p].at[pl.ds(kk * B, B)], sem_b[p])

    def compute(c, p):
        @pl.when(c < n)
        def _():
            @pl.when(c >= 1)
            def _():
                pltpu.make_async_copy(
                    out_hbm.at[pl.ds(0, CE * H)], ob, sem_out).wait()
            pltpu.make_async_copy(
                ta_hbm.at[pl.ds(0, CE)], ra.at[p], sem_a[p]).wait()
            pltpu.make_async_copy(
                tb_hbm.at[pl.ds(0, CE)], rb.at[p], sem_b[p]).wait()

            def pair_body(i, carry2):
                a0 = ra[p, 2 * i, :] + rb[p, 2 * i, :]
                a1 = ra[p, 2 * i + 1, :] + rb[p, 2 * i + 1, :]
                s = jnp.where(lane_lo, a0, a1)
                s = jnp.clip(s, -30.0, 30.0)
                d = 1.0 + jnp.exp(s)
                y = plsc.bitcast(
                    jnp.int32(0x7EF127EA) - plsc.bitcast(d, jnp.int32),
                    jnp.float32)
                y = y * (2.0 - d * y)
                w = y * (2.0 - d * y)
                off = lax.shift_right_logical(i, 6) * (H * B) + (i & 63) * 2
                plsc.store_scatter(ob, [svec + off], w)
                return carry2

            lax.fori_loop(0, CE // 2, pair_body, 0, unroll=8)
            obase = (wid + c * NW) * (CE * H)
            pltpu.async_copy(ob, out_hbm.at[pl.ds(obase, CE * H)], sem_out)

    fire(0, 0)

    def body(q, carry):
        c0 = 2 * q
        fire(c0 + 1, 1)
        compute(c0, 0)
        fire(c0 + 2, 0)
        compute(c0 + 1, 1)
        return carry

    lax.fori_loop(0, (BASE_CHUNKS + 2) // 2, body, 0)
    pltpu.make_async_copy(out_hbm.at[pl.ds(0, CE * H)], ob, sem_out).wait()


def kernel(concept_hidden, head, tail, W, scoring_fn_source, scoring_fn_target):
    x = concept_hidden.astype(jnp.float32)
    hsel = (jnp.arange(IN) // OUT)[:, None] == jnp.arange(H)[None, :]
    onehot = hsel.astype(jnp.float32)
    s_src = -scoring_fn_source.reshape(IN)[:, None] * onehot
    s_tgt = -scoring_fn_target.reshape(IN)[:, None] * onehot
    S_a = jnp.concatenate([s_src, s_src], axis=1)
    S_b = jnp.concatenate([s_tgt, s_tgt], axis=1)

    table_a, table_b = _make_tables(x, W.astype(jnp.float32), S_a, S_b)

    head2d = head.astype(jnp.int32).reshape(NB, B)
    tail2d = tail.astype(jnp.int32).reshape(NB, B)
    out_flat = _sc_edges(table_a, table_b, head2d, tail2d)
    return out_flat.reshape(NB, H, B).transpose(0, 2, 1).reshape(E, H)


# speedup vs baseline: 2.4619x; 2.4619x over previous
import functools

import jax
import jax.numpy as jnp
from jax import lax
from jax.experimental import pallas as pl
from jax.experimental.pallas import tpu as pltpu
from jax.experimental.pallas import tpu_sc as plsc

N = 10000
E = 320000
IN = 128
H = 8
OUT = 16

NC = 2
NS = 16
NW = NC * NS
B = 128
NB = E // B
CB = 10
NCHUNKS = NB // CB
CE = CB * B
BASE_CHUNKS = NCHUNKS // NW
EXTRA = NCHUNKS - BASE_CHUNKS * NW


def _tc_tables(x_ref, w_ref, sa_ref, sb_ref, a_ref, b_ref):
    wa = jnp.dot(w_ref[...], sa_ref[...], preferred_element_type=jnp.float32)
    wb = jnp.dot(w_ref[...], sb_ref[...], preferred_element_type=jnp.float32)
    x = x_ref[...]
    a_ref[...] = jnp.dot(x, wa, preferred_element_type=jnp.float32)
    b_ref[...] = jnp.dot(x, wb, preferred_element_type=jnp.float32)


def _make_tables(x, W, S_a, S_b):
    return pl.pallas_call(
        _tc_tables,
        out_shape=(
            jax.ShapeDtypeStruct((N, 16), jnp.float32),
            jax.ShapeDtypeStruct((N, 16), jnp.float32),
        ),
    )(x, W, S_a, S_b)


_sc_mesh = plsc.VectorSubcoreMesh(
    core_axis_name="c", subcore_axis_name="s", num_cores=NC, num_subcores=NS
)


@functools.partial(
    pl.kernel,
    out_type=jax.ShapeDtypeStruct((E * H,), jnp.float32),
    mesh=_sc_mesh,
    scratch_types=[
        pltpu.VMEM((2, CB, B), jnp.int32),
        pltpu.VMEM((2, CB, B), jnp.int32),
        pltpu.VMEM((2, CE, 16), jnp.float32),
        pltpu.VMEM((2, CE, 16), jnp.float32),
        pltpu.VMEM((CE * H,), jnp.float32),
        pltpu.SemaphoreType.DMA,
        pltpu.SemaphoreType.DMA,
        pltpu.SemaphoreType.DMA,
        pltpu.SemaphoreType.DMA,
        pltpu.SemaphoreType.DMA,
    ],
    compiler_params=pltpu.CompilerParams(
        use_tc_tiling_on_sc=False, needs_layout_passes=False),
)
def _sc_edges(ta_hbm, tb_hbm, head_hbm, tail_hbm, out_hbm,
              hidx, tidx, ra, rb, ob,
              sem_a0, sem_a1, sem_b0, sem_b1, sem_out):
    wid = lax.axis_index("s") * NC + lax.axis_index("c")
    n = jnp.where(wid < EXTRA, BASE_CHUNKS + 1, BASE_CHUNKS)
    lane = lax.iota(jnp.int32, 16)
    lane_lo = lane < 8
    svec = (lane & 7) * B + lax.shift_right_logical(lane, 3)
    sem_a = (sem_a0, sem_a1)
    sem_b = (sem_b0, sem_b1)

    def fire(c, p):
        @pl.when(c < n)
        def _():
            rbase = (wid + c * NW) * CB
            pltpu.sync_copy(head_hbm.at[pl.ds(rbase, CB)], hidx.at[p])
            pltpu.sync_copy(tail_hbm.at[pl.ds(rbase, CB)], tidx.at[p])
            for kk in range(CB):
                pltpu.async_copy(ta_hbm.at[hidx.at[p].at[kk]],
                                 ra.at[p].at[pl.ds(kk * B, B)], sem_a[p])
                pltpu.async_copy(tb_hbm.at[tidx.at[p].at[kk]],
                                 rb.at[p].at[pl.ds(kk * B, B)], sem_b[p])

    def compute(c, p):
        @pl.when(c < n)
        def _():
            @pl.when(c >= 1)
            def _():
                pltpu.make_async_copy(
                    out_hbm.at[pl.ds(0, CE * H)], ob, sem_out).wait()
            pltpu.make_async_copy(
                ta_hbm.at[pl.ds(0, CE)], ra.at[p], sem_a[p]).wait()
            pltpu.make_async_copy(
                tb_hbm.at[pl.ds(0, CE)], rb.at[p], sem_b[p]).wait()

            @plsc.parallel_loop(0, CE // 2, unroll=8)
            def pair_body(i):
                a0 = ra[p, 2 * i, :] + rb[p, 2 * i, :]
                a1 = ra[p, 2 * i + 1, :] + rb[p, 2 * i + 1, :]
                s = jnp.where(lane_lo, a0, a1)
                w = 1.0 / (1.0 + jnp.exp(s))
                off = lax.shift_right_logical(i, 6) * (H * B) + (i & 63) * 2
                plsc.store_scatter(ob, [svec + off], w)
            obase = (wid + c * NW) * (CE * H)
            pltpu.async_copy(ob, out_hbm.at[pl.ds(obase, CE * H)], sem_out)

    fire(0, 0)

    def body(q, carry):
        c0 = 2 * q
        fire(c0 + 1, 1)
        compute(c0, 0)
        fire(c0 + 2, 0)
        compute(c0 + 1, 1)
        return carry

    lax.fori_loop(0, (BASE_CHUNKS + 2) // 2, body, 0)
    pltpu.make_async_copy(out_hbm.at[pl.ds(0, CE * H)], ob, sem_out).wait()


def kernel(concept_hidden, head, tail, W, scoring_fn_source, scoring_fn_target):
    x = concept_hidden.astype(jnp.float32)
    hsel = (jnp.arange(IN) // OUT)[:, None] == jnp.arange(H)[None, :]
    onehot = hsel.astype(jnp.float32)
    s_src = -scoring_fn_source.reshape(IN)[:, None] * onehot
    s_tgt = -scoring_fn_target.reshape(IN)[:, None] * onehot
    S_a = jnp.concatenate([s_src, s_src], axis=1)
    S_b = jnp.concatenate([s_tgt, s_tgt], axis=1)

    table_a, table_b = _make_tables(x, W.astype(jnp.float32), S_a, S_b)

    head2d = head.astype(jnp.int32).reshape(NB, B)
    tail2d = tail.astype(jnp.int32).reshape(NB, B)
    out_flat = _sc_edges(table_a, table_b, head2d, tail2d)
    return out_flat.reshape(NB, H, B).transpose(0, 2, 1).reshape(E, H)


# speedup vs baseline: 2.7829x; 1.1304x over previous
import functools

import jax
import jax.numpy as jnp
from jax import lax
from jax.experimental import pallas as pl
from jax.experimental.pallas import tpu as pltpu
from jax.experimental.pallas import tpu_sc as plsc

N = 10000
E = 320000
IN = 128
H = 8
OUT = 16

NC = 2
NS = 16
NW = NC * NS
B = 128
NB = E // B
CB = 10
NCHUNKS = NB // CB
CE = CB * B
BASE_CHUNKS = NCHUNKS // NW
EXTRA = NCHUNKS - BASE_CHUNKS * NW


def _tc_tables(x_ref, w_ref, sa_ref, sb_ref, a_ref, b_ref):
    wa = jnp.dot(w_ref[...], sa_ref[...], preferred_element_type=jnp.float32)
    wb = jnp.dot(w_ref[...], sb_ref[...], preferred_element_type=jnp.float32)
    x = x_ref[...]
    ta = jnp.dot(x, wa, preferred_element_type=jnp.float32)
    tb = jnp.dot(x, wb, preferred_element_type=jnp.float32)
    a_ref[:, 0:16] = ta
    b_ref[:, 0:16] = tb


def _make_tables(x, W, S_a, S_b):
    return pl.pallas_call(
        _tc_tables,
        out_shape=(
            jax.ShapeDtypeStruct((N, 128), jnp.float32),
            jax.ShapeDtypeStruct((N, 128), jnp.float32),
        ),
    )(x, W, S_a, S_b)


_sc_mesh = plsc.VectorSubcoreMesh(
    core_axis_name="c", subcore_axis_name="s", num_cores=NC, num_subcores=NS
)


@functools.partial(
    pl.kernel,
    out_type=jax.ShapeDtypeStruct((E * H,), jnp.float32),
    mesh=_sc_mesh,
    scratch_types=[
        pltpu.VMEM((2, CB, B), jnp.int32),
        pltpu.VMEM((2, CB, B), jnp.int32),
        pltpu.VMEM((2, CE, 16), jnp.float32),
        pltpu.VMEM((2, CE, 16), jnp.float32),
        pltpu.VMEM((CE * H,), jnp.float32),
        pltpu.VMEM_SHARED((N, 16), jnp.float32),
        pltpu.VMEM_SHARED((N, 16), jnp.float32),
        pltpu.SemaphoreType.DMA,
        pltpu.SemaphoreType.DMA,
        pltpu.SemaphoreType.DMA,
        pltpu.SemaphoreType.DMA,
        pltpu.SemaphoreType.DMA,
    ],
    compiler_params=pltpu.CompilerParams(
        use_tc_tiling_on_sc=False, needs_layout_passes=False),
)
def _sc_edges(ta_hbm, tb_hbm, head_hbm, tail_hbm, out_hbm,
              hidx, tidx, ra, rb, ob, spa, spb,
              sem_a0, sem_a1, sem_b0, sem_b1, sem_out):
    sid = lax.axis_index("s")
    wid = sid * NC + lax.axis_index("c")
    rows_per_sid = N // NS
    pltpu.sync_copy(
        ta_hbm.at[pl.ds(sid * rows_per_sid, rows_per_sid), pl.ds(0, 16)],
        spa.at[pl.ds(sid * rows_per_sid, rows_per_sid)])
    pltpu.sync_copy(
        tb_hbm.at[pl.ds(sid * rows_per_sid, rows_per_sid), pl.ds(0, 16)],
        spb.at[pl.ds(sid * rows_per_sid, rows_per_sid)])
    plsc.subcore_barrier()
    n = jnp.where(wid < EXTRA, BASE_CHUNKS + 1, BASE_CHUNKS)
    lane = lax.iota(jnp.int32, 16)
    lane_lo = lane < 8
    svec = (lane & 7) * B + lax.shift_right_logical(lane, 3)
    sem_a = (sem_a0, sem_a1)
    sem_b = (sem_b0, sem_b1)

    def fire(c, p):
        @pl.when(c < n)
        def _():
            rbase = (wid + c * NW) * CB
            pltpu.sync_copy(head_hbm.at[pl.ds(rbase, CB)], hidx.at[p])
            pltpu.sync_copy(tail_hbm.at[pl.ds(rbase, CB)], tidx.at[p])
            for kk in range(CB):
                pltpu.async_copy(spa.at[hidx.at[p].at[kk]],
                                 ra.at[p].at[pl.ds(kk * B, B)], sem_a[p])
                pltpu.async_copy(spb.at[tidx.at[p].at[kk]],
                                 rb.at[p].at[pl.ds(kk * B, B)], sem_b[p])

    def compute(c, p):
        @pl.when(c < n)
        def _():
            @pl.when(c >= 1)
            def _():
                pltpu.make_async_copy(
                    out_hbm.at[pl.ds(0, CE * H)], ob, sem_out).wait()
            pltpu.make_async_copy(
                spa.at[pl.ds(0, CE)], ra.at[p], sem_a[p]).wait()
            pltpu.make_async_copy(
                spb.at[pl.ds(0, CE)], rb.at[p], sem_b[p]).wait()

            @plsc.parallel_loop(0, CE // 2, unroll=8)
            def pair_body(i):
                a0 = ra[p, 2 * i, :] + rb[p, 2 * i, :]
                a1 = ra[p, 2 * i + 1, :] + rb[p, 2 * i + 1, :]
                s = jnp.where(lane_lo, a0, a1)
                w = 1.0 / (1.0 + jnp.exp(s))
                off = lax.shift_right_logical(i, 6) * (H * B) + (i & 63) * 2
                plsc.store_scatter(ob, [svec + off], w)
            obase = (wid + c * NW) * (CE * H)
            pltpu.async_copy(ob, out_hbm.at[pl.ds(obase, CE * H)], sem_out)

    fire(0, 0)

    def body(q, carry):
        c0 = 2 * q
        fire(c0 + 1, 1)
        compute(c0, 0)
        fire(c0 + 2, 0)
        compute(c0 + 1, 1)
        return carry

    lax.fori_loop(0, (BASE_CHUNKS + 2) // 2, body, 0)
    pltpu.make_async_copy(out_hbm.at[pl.ds(0, CE * H)], ob, sem_out).wait()


def kernel(concept_hidden, head, tail, W, scoring_fn_source, scoring_fn_target):
    x = concept_hidden.astype(jnp.float32)
    hsel = (jnp.arange(IN) // OUT)[:, None] == jnp.arange(H)[None, :]
    onehot = hsel.astype(jnp.float32)
    s_src = -scoring_fn_source.reshape(IN)[:, None] * onehot
    s_tgt = -scoring_fn_target.reshape(IN)[:, None] * onehot
    S_a = jnp.concatenate([s_src, s_src], axis=1)
    S_b = jnp.concatenate([s_tgt, s_tgt], axis=1)

    table_a, table_b = _make_tables(x, W.astype(jnp.float32), S_a, S_b)

    head2d = head.astype(jnp.int32).reshape(NB, B)
    tail2d = tail.astype(jnp.int32).reshape(NB, B)
    out_flat = _sc_edges(table_a, table_b, head2d, tail2d)
    return out_flat.reshape(NB, H, B).transpose(0, 2, 1).reshape(E, H)
